# baseline (speedup 1.0000x reference)
import functools

import jax
import jax.numpy as jnp
from jax import lax
from jax.experimental import pallas as pl

_SLOPE = 0.2
_N_HEADS = 4
_LOG2E = 1.4426950408889634

_DN_COL = (((1,), (1,)), ((), ()))


def _proj_kernel(x_ref, w_ref, asrc_ref, adst_ref, h_ref, f1_ref, f2_ref):
    xx = x_ref[...]
    for hi in range(_N_HEADS):
        h = jnp.dot(xx, w_ref[hi], preferred_element_type=jnp.float32)
        h_ref[hi] = h
        f1_ref[hi] = lax.dot_general(h, asrc_ref[hi][None, :] * _LOG2E,
                                     _DN_COL, preferred_element_type=jnp.float32)
        f2_ref[hi] = lax.dot_general(adst_ref[hi][None, :] * _LOG2E, h,
                                     _DN_COL, preferred_element_type=jnp.float32)


def _attn_body(h_all_ref, adj_ref, x_ref, f1_ref, f2_ref):
    x_blk = x_ref[...]
    adj_blk = adj_ref[...]
    acc = jnp.zeros_like(x_blk)
    for hi in range(_N_HEADS):
        h_full = h_all_ref[hi]
        e = f1_ref[hi] + f2_ref[hi]
        e = jnp.maximum(e, _SLOPE * e)
        p = jnp.exp2(e) * adj_blk
        s = jnp.sum(p, axis=-1, keepdims=True)
        out = jnp.dot(p, h_full, preferred_element_type=jnp.float32) / s
        v = out + x_blk
        acc = acc + jnp.where(v > 0, v, jnp.exp(jnp.minimum(v, 0.0)) - 1.0)
    return acc * (1.0 / _N_HEADS)


def _attn_mid_kernel(h_all_ref, adj_ref, x_ref, f1_ref, f2_ref,
                     w_ref, asrc_ref, adst_ref,
                     y_ref, hn_ref, f1n_ref, f2n_ref):
    y = _attn_body(h_all_ref, adj_ref, x_ref, f1_ref, f2_ref)
    y_ref[...] = y
    for hi in range(_N_HEADS):
        hn = jnp.dot(y, w_ref[hi], preferred_element_type=jnp.float32)
        hn_ref[hi] = hn
        f1n_ref[hi] = lax.dot_general(hn, asrc_ref[hi][None, :] * _LOG2E,
                                      _DN_COL, preferred_element_type=jnp.float32)
        f2n_ref[hi] = lax.dot_general(adst_ref[hi][None, :] * _LOG2E, hn,
                                      _DN_COL, preferred_element_type=jnp.float32)


def _attn_last_kernel(h_all_ref, adj_ref, x_ref, f1_ref, f2_ref, y_ref):
    y_ref[...] = _attn_body(h_all_ref, adj_ref, x_ref, f1_ref, f2_ref)


def _gram_kernel(y_all_ref, y_blk_ref, out_ref):
    out_ref[...] = lax.dot_general(y_all_ref[...], y_blk_ref[...],
                                   _DN_COL, preferred_element_type=jnp.float32)


@functools.partial(jax.jit, static_argnames=())
def kernel(x, adj, W, a_src, a_dst):
    N, D = x.shape
    H = W.shape[0]
    B = 256
    GB = 512

    proj = pl.pallas_call(
        _proj_kernel,
        out_shape=(
            jax.ShapeDtypeStruct((H, N, D), jnp.float32),
            jax.ShapeDtypeStruct((H, N, 1), jnp.float32),
            jax.ShapeDtypeStruct((H, 1, N), jnp.float32),
        ),
    )

    attn_in_specs = [
        pl.BlockSpec((H, N, D), lambda i: (0, 0, 0)),
        pl.BlockSpec((B, N), lambda i: (i, 0)),
        pl.BlockSpec((B, D), lambda i: (i, 0)),
        pl.BlockSpec((H, B, 1), lambda i: (0, i, 0)),
        pl.BlockSpec((H, 1, N), lambda i: (0, 0, 0)),
    ]

    attn_mid = pl.pallas_call(
        _attn_mid_kernel,
        grid=(N // B,),
        in_specs=attn_in_specs + [
            pl.BlockSpec((H, D, D), lambda i: (0, 0, 0)),
            pl.BlockSpec((H, D), lambda i: (0, 0)),
            pl.BlockSpec((H, D), lambda i: (0, 0)),
        ],
        out_specs=(
            pl.BlockSpec((B, D), lambda i: (i, 0)),
            pl.BlockSpec((H, B, D), lambda i: (0, i, 0)),
            pl.BlockSpec((H, B, 1), lambda i: (0, i, 0)),
            pl.BlockSpec((H, 1, B), lambda i: (0, 0, i)),
        ),
        out_shape=(
            jax.ShapeDtypeStruct((N, D), jnp.float32),
            jax.ShapeDtypeStruct((H, N, D), jnp.float32),
            jax.ShapeDtypeStruct((H, N, 1), jnp.float32),
            jax.ShapeDtypeStruct((H, 1, N), jnp.float32),
        ),
    )

    attn_last = pl.pallas_call(
        _attn_last_kernel,
        grid=(N // B,),
        in_specs=attn_in_specs,
        out_specs=pl.BlockSpec((B, D), lambda i: (i, 0)),
        out_shape=jax.ShapeDtypeStruct((N, D), jnp.float32),
    )

    gram = pl.pallas_call(
        _gram_kernel,
        grid=(N // GB,),
        in_specs=[
            pl.BlockSpec((N, D), lambda i: (0, 0)),
            pl.BlockSpec((GB, D), lambda i: (i, 0)),
        ],
        out_specs=pl.BlockSpec((N, GB), lambda i: (0, i)),
        out_shape=jax.ShapeDtypeStruct((N, N), jnp.float32),
    )

    h, f1, f2 = proj(x, W, a_src, a_dst)
    y1, h, f1, f2 = attn_mid(h, adj, x, f1, f2, W, a_src, a_dst)
    y2, h, f1, f2 = attn_mid(h, adj, y1, f1, f2, W, a_src, a_dst)
    y3 = attn_last(h, adj, y2, f1, f2)
    return gram(y3, y3)


# speedup vs baseline: 1.0829x; 1.0829x over previous
import functools

import jax
import jax.numpy as jnp
from jax import lax
from jax.experimental import pallas as pl

_SLOPE = 0.2
_N_HEADS = 4
_LOG2E = 1.4426950408889634

_DN_COL = (((1,), (1,)), ((), ()))


def _proj_kernel(x_ref, w_ref, asrc_ref, adst_ref, h_ref, f1_ref, f2_ref):
    xx = x_ref[...]
    for hi in range(_N_HEADS):
        h = jnp.dot(xx, w_ref[hi], preferred_element_type=jnp.float32)
        h_ref[hi] = h
        f1_ref[hi] = lax.dot_general(h, asrc_ref[hi][None, :] * _LOG2E,
                                     _DN_COL, preferred_element_type=jnp.float32)
        f2_ref[hi] = lax.dot_general(adst_ref[hi][None, :] * _LOG2E, h,
                                     _DN_COL, preferred_element_type=jnp.float32)


_CHUNK = 1024


def _attn_body(h_all_ref, adj_ref, x_ref, f1_ref, f2_ref):
    x_blk = x_ref[...]
    n = adj_ref.shape[-1]
    acc = jnp.zeros_like(x_blk)
    for hi in range(_N_HEADS):
        f1 = f1_ref[hi]
        out = jnp.zeros_like(x_blk)
        s = jnp.zeros_like(f1)
        for c in range(0, n, _CHUNK):
            e = f1 + f2_ref[hi, :, c:c + _CHUNK]
            e = jnp.maximum(e, _SLOPE * e)
            p = jnp.exp2(e) * adj_ref[:, c:c + _CHUNK]
            s = s + jnp.sum(p, axis=-1, keepdims=True)
            out = out + jnp.dot(p, h_all_ref[hi, c:c + _CHUNK, :],
                                preferred_element_type=jnp.float32)
        v = out / s + x_blk
        acc = acc + jnp.where(v > 0, v, jnp.exp(jnp.minimum(v, 0.0)) - 1.0)
    return acc * (1.0 / _N_HEADS)


def _attn_mid_kernel(h_all_ref, adj_ref, x_ref, f1_ref, f2_ref,
                     w_ref, asrc_ref, adst_ref,
                     y_ref, hn_ref, f1n_ref, f2n_ref):
    y = _attn_body(h_all_ref, adj_ref, x_ref, f1_ref, f2_ref)
    y_ref[...] = y
    for hi in range(_N_HEADS):
        hn = jnp.dot(y, w_ref[hi], preferred_element_type=jnp.float32)
        hn_ref[hi] = hn
        f1n_ref[hi] = lax.dot_general(hn, asrc_ref[hi][None, :] * _LOG2E,
                                      _DN_COL, preferred_element_type=jnp.float32)
        f2n_ref[hi] = lax.dot_general(adst_ref[hi][None, :] * _LOG2E, hn,
                                      _DN_COL, preferred_element_type=jnp.float32)


def _attn_last_kernel(h_all_ref, adj_ref, x_ref, f1_ref, f2_ref, y_ref):
    y_ref[...] = _attn_body(h_all_ref, adj_ref, x_ref, f1_ref, f2_ref)


def _gram_kernel(y_all_ref, y_blk_ref, out_ref):
    out_ref[...] = lax.dot_general(y_all_ref[...], y_blk_ref[...],
                                   _DN_COL, preferred_element_type=jnp.float32)


@functools.partial(jax.jit, static_argnames=())
def kernel(x, adj, W, a_src, a_dst):
    N, D = x.shape
    H = W.shape[0]
    B = 512
    GB = 512

    proj = pl.pallas_call(
        _proj_kernel,
        out_shape=(
            jax.ShapeDtypeStruct((H, N, D), jnp.float32),
            jax.ShapeDtypeStruct((H, N, 1), jnp.float32),
            jax.ShapeDtypeStruct((H, 1, N), jnp.float32),
        ),
    )

    attn_in_specs = [
        pl.BlockSpec((H, N, D), lambda i: (0, 0, 0)),
        pl.BlockSpec((B, N), lambda i: (i, 0)),
        pl.BlockSpec((B, D), lambda i: (i, 0)),
        pl.BlockSpec((H, B, 1), lambda i: (0, i, 0)),
        pl.BlockSpec((H, 1, N), lambda i: (0, 0, 0)),
    ]

    attn_mid = pl.pallas_call(
        _attn_mid_kernel,
        grid=(N // B,),
        in_specs=attn_in_specs + [
            pl.BlockSpec((H, D, D), lambda i: (0, 0, 0)),
            pl.BlockSpec((H, D), lambda i: (0, 0)),
            pl.BlockSpec((H, D), lambda i: (0, 0)),
        ],
        out_specs=(
            pl.BlockSpec((B, D), lambda i: (i, 0)),
            pl.BlockSpec((H, B, D), lambda i: (0, i, 0)),
            pl.BlockSpec((H, B, 1), lambda i: (0, i, 0)),
            pl.BlockSpec((H, 1, B), lambda i: (0, 0, i)),
        ),
        out_shape=(
            jax.ShapeDtypeStruct((N, D), jnp.float32),
            jax.ShapeDtypeStruct((H, N, D), jnp.float32),
            jax.ShapeDtypeStruct((H, N, 1), jnp.float32),
            jax.ShapeDtypeStruct((H, 1, N), jnp.float32),
        ),
    )

    attn_last = pl.pallas_call(
        _attn_last_kernel,
        grid=(N // B,),
        in_specs=attn_in_specs,
        out_specs=pl.BlockSpec((B, D), lambda i: (i, 0)),
        out_shape=jax.ShapeDtypeStruct((N, D), jnp.float32),
    )

    gram = pl.pallas_call(
        _gram_kernel,
        grid=(N // GB,),
        in_specs=[
            pl.BlockSpec((N, D), lambda i: (0, 0)),
            pl.BlockSpec((GB, D), lambda i: (i, 0)),
        ],
        out_specs=pl.BlockSpec((N, GB), lambda i: (0, i)),
        out_shape=jax.ShapeDtypeStruct((N, N), jnp.float32),
    )

    h, f1, f2 = proj(x, W, a_src, a_dst)
    y1, h, f1, f2 = attn_mid(h, adj, x, f1, f2, W, a_src, a_dst)
    y2, h, f1, f2 = attn_mid(h, adj, y1, f1, f2, W, a_src, a_dst)
    y3 = attn_last(h, adj, y2, f1, f2)
    return gram(y3, y3)


# speedup vs baseline: 1.1079x; 1.0231x over previous
import functools

import jax
import jax.numpy as jnp
from jax import lax
from jax.experimental import pallas as pl

_SLOPE = 0.2
_N_HEADS = 4
_LOG2E = 1.4426950408889634

_DN_COL = (((1,), (1,)), ((), ()))


def _proj_kernel(x_ref, w_ref, asrc_ref, adst_ref, h_ref, f1_ref, f2_ref):
    xx = x_ref[...]
    for hi in range(_N_HEADS):
        h = jnp.dot(xx, w_ref[hi], preferred_element_type=jnp.float32)
        h_ref[hi] = h
        f1_ref[hi] = lax.dot_general(h, asrc_ref[hi][None, :] * _LOG2E,
                                     _DN_COL, preferred_element_type=jnp.float32)
        f2_ref[hi] = lax.dot_general(adst_ref[hi][None, :] * _LOG2E, h,
                                     _DN_COL, preferred_element_type=jnp.float32)


_CHUNK = 512


def _attn_body(h_all_ref, adj_ref, x_ref, f1_ref, f2_ref):
    x_blk = x_ref[...]
    n = adj_ref.shape[-1]
    acc = jnp.zeros_like(x_blk)
    for hi in range(_N_HEADS):
        f1 = f1_ref[hi]
        out = jnp.zeros_like(x_blk)
        s = jnp.zeros_like(f1)
        for c in range(0, n, _CHUNK):
            e = f1 + f2_ref[hi, :, c:c + _CHUNK]
            e = jnp.maximum(e, _SLOPE * e)
            p = jnp.exp2(e) * adj_ref[:, c:c + _CHUNK]
            s = s + jnp.sum(p, axis=-1, keepdims=True)
            out = out + jnp.dot(p, h_all_ref[hi, c:c + _CHUNK, :],
                                preferred_element_type=jnp.float32)
        v = out / s + x_blk
        acc = acc + jnp.where(v > 0, v, jnp.exp(jnp.minimum(v, 0.0)) - 1.0)
    return acc * (1.0 / _N_HEADS)


def _attn_mid_kernel(h_all_ref, adj_ref, x_ref, f1_ref, f2_ref,
                     w_ref, asrc_ref, adst_ref,
                     y_ref, hn_ref, f1n_ref, f2n_ref):
    y = _attn_body(h_all_ref, adj_ref, x_ref, f1_ref, f2_ref)
    y_ref[...] = y
    for hi in range(_N_HEADS):
        hn = jnp.dot(y, w_ref[hi], preferred_element_type=jnp.float32)
        hn_ref[hi] = hn
        f1n_ref[hi] = lax.dot_general(hn, asrc_ref[hi][None, :] * _LOG2E,
                                      _DN_COL, preferred_element_type=jnp.float32)
        f2n_ref[hi] = lax.dot_general(adst_ref[hi][None, :] * _LOG2E, hn,
                                      _DN_COL, preferred_element_type=jnp.float32)


def _attn_last_kernel(h_all_ref, adj_ref, x_ref, f1_ref, f2_ref, y_ref):
    y_ref[...] = _attn_body(h_all_ref, adj_ref, x_ref, f1_ref, f2_ref)


def _gram_kernel(y_all_ref, y_blk_ref, out_ref):
    out_ref[...] = lax.dot_general(y_all_ref[...], y_blk_ref[...],
                                   _DN_COL, preferred_element_type=jnp.float32)


@functools.partial(jax.jit, static_argnames=())
def kernel(x, adj, W, a_src, a_dst):
    N, D = x.shape
    H = W.shape[0]
    B = 512
    GB = 512

    proj = pl.pallas_call(
        _proj_kernel,
        out_shape=(
            jax.ShapeDtypeStruct((H, N, D), jnp.float32),
            jax.ShapeDtypeStruct((H, N, 1), jnp.float32),
            jax.ShapeDtypeStruct((H, 1, N), jnp.float32),
        ),
    )

    attn_in_specs = [
        pl.BlockSpec((H, N, D), lambda i: (0, 0, 0)),
        pl.BlockSpec((B, N), lambda i: (i, 0)),
        pl.BlockSpec((B, D), lambda i: (i, 0)),
        pl.BlockSpec((H, B, 1), lambda i: (0, i, 0)),
        pl.BlockSpec((H, 1, N), lambda i: (0, 0, 0)),
    ]

    attn_mid = pl.pallas_call(
        _attn_mid_kernel,
        grid=(N // B,),
        in_specs=attn_in_specs + [
            pl.BlockSpec((H, D, D), lambda i: (0, 0, 0)),
            pl.BlockSpec((H, D), lambda i: (0, 0)),
            pl.BlockSpec((H, D), lambda i: (0, 0)),
        ],
        out_specs=(
            pl.BlockSpec((B, D), lambda i: (i, 0)),
            pl.BlockSpec((H, B, D), lambda i: (0, i, 0)),
            pl.BlockSpec((H, B, 1), lambda i: (0, i, 0)),
            pl.BlockSpec((H, 1, B), lambda i: (0, 0, i)),
        ),
        out_shape=(
            jax.ShapeDtypeStruct((N, D), jnp.float32),
            jax.ShapeDtypeStruct((H, N, D), jnp.float32),
            jax.ShapeDtypeStruct((H, N, 1), jnp.float32),
            jax.ShapeDtypeStruct((H, 1, N), jnp.float32),
        ),
    )

    attn_last = pl.pallas_call(
        _attn_last_kernel,
        grid=(N // B,),
        in_specs=attn_in_specs,
        out_specs=pl.BlockSpec((B, D), lambda i: (i, 0)),
        out_shape=jax.ShapeDtypeStruct((N, D), jnp.float32),
    )

    gram = pl.pallas_call(
        _gram_kernel,
        grid=(N // GB,),
        in_specs=[
            pl.BlockSpec((N, D), lambda i: (0, 0)),
            pl.BlockSpec((GB, D), lambda i: (i, 0)),
        ],
        out_specs=pl.BlockSpec((N, GB), lambda i: (0, i)),
        out_shape=jax.ShapeDtypeStruct((N, N), jnp.float32),
    )

    h, f1, f2 = proj(x, W, a_src, a_dst)
    y1, h, f1, f2 = attn_mid(h, adj, x, f1, f2, W, a_src, a_dst)
    y2, h, f1, f2 = attn_mid(h, adj, y1, f1, f2, W, a_src, a_dst)
    y3 = attn_last(h, adj, y2, f1, f2)
    return gram(y3, y3)


# speedup vs baseline: 1.3031x; 1.1762x over previous
import functools

import jax
import jax.numpy as jnp
from jax import lax
from jax.experimental import pallas as pl

_SLOPE = 0.2
_N_HEADS = 4
_LOG2E = 1.4426950408889634

_DN_COL = (((1,), (1,)), ((), ()))


def _proj_kernel(x_ref, w_ref, asrc_ref, adst_ref, h_ref, f1_ref, f2_ref):
    xx = x_ref[...]
    for hi in range(_N_HEADS):
        h = jnp.dot(xx, w_ref[hi], preferred_element_type=jnp.float32)
        h_ref[hi] = h
        f1_ref[hi] = lax.dot_general(h, asrc_ref[hi][None, :] * _LOG2E,
                                     _DN_COL, preferred_element_type=jnp.float32)
        f2_ref[hi] = lax.dot_general(adst_ref[hi][None, :] * _LOG2E, h,
                                     _DN_COL, preferred_element_type=jnp.float32)


_CHUNK = 256


def _attn_body(h_all_ref, adj_ref, x_ref, f1_ref, f2_ref):
    x_blk = x_ref[...]
    n = adj_ref.shape[-1]
    acc = jnp.zeros_like(x_blk)
    for hi in range(_N_HEADS):
        f1 = f1_ref[hi]
        out = jnp.zeros_like(x_blk)
        s = jnp.zeros_like(f1)
        for c in range(0, n, _CHUNK):
            e = f1 + f2_ref[hi, :, c:c + _CHUNK]
            e = jnp.maximum(e, _SLOPE * e)
            p = jnp.exp2(e) * adj_ref[:, c:c + _CHUNK]
            s = s + jnp.sum(p, axis=-1, keepdims=True)
            out = out + jnp.dot(p, h_all_ref[hi, c:c + _CHUNK, :],
                                preferred_element_type=jnp.float32)
        v = out / s + x_blk
        acc = acc + jnp.where(v > 0, v, jnp.exp(jnp.minimum(v, 0.0)) - 1.0)
    return acc * (1.0 / _N_HEADS)


def _attn_mid_kernel(h_all_ref, adj_ref, x_ref, f1_ref, f2_ref,
                     w_ref, asrc_ref, adst_ref,
                     y_ref, hn_ref, f1n_ref, f2n_ref):
    y = _attn_body(h_all_ref, adj_ref, x_ref, f1_ref, f2_ref)
    y_ref[...] = y
    for hi in range(_N_HEADS):
        hn = jnp.dot(y, w_ref[hi], preferred_element_type=jnp.float32)
        hn_ref[hi] = hn
        f1n_ref[hi] = lax.dot_general(hn, asrc_ref[hi][None, :] * _LOG2E,
                                      _DN_COL, preferred_element_type=jnp.float32)
        f2n_ref[hi] = lax.dot_general(adst_ref[hi][None, :] * _LOG2E, hn,
                                      _DN_COL, preferred_element_type=jnp.float32)


def _attn_last_kernel(h_all_ref, adj_ref, x_ref, f1_ref, f2_ref, y_ref):
    y_ref[...] = _attn_body(h_all_ref, adj_ref, x_ref, f1_ref, f2_ref)


def _gram_kernel(y_all_ref, y_blk_ref, out_ref):
    out_ref[...] = lax.dot_general(y_all_ref[...], y_blk_ref[...],
                                   _DN_COL, preferred_element_type=jnp.float32)


@functools.partial(jax.jit, static_argnames=())
def kernel(x, adj, W, a_src, a_dst):
    N, D = x.shape
    H = W.shape[0]
    B = 512
    GB = 512

    proj = pl.pallas_call(
        _proj_kernel,
        out_shape=(
            jax.ShapeDtypeStruct((H, N, D), jnp.float32),
            jax.ShapeDtypeStruct((H, N, 1), jnp.float32),
            jax.ShapeDtypeStruct((H, 1, N), jnp.float32),
        ),
    )

    attn_in_specs = [
        pl.BlockSpec((H, N, D), lambda i: (0, 0, 0)),
        pl.BlockSpec((B, N), lambda i: (i, 0)),
        pl.BlockSpec((B, D), lambda i: (i, 0)),
        pl.BlockSpec((H, B, 1), lambda i: (0, i, 0)),
        pl.BlockSpec((H, 1, N), lambda i: (0, 0, 0)),
    ]

    attn_mid = pl.pallas_call(
        _attn_mid_kernel,
        grid=(N // B,),
        in_specs=attn_in_specs + [
            pl.BlockSpec((H, D, D), lambda i: (0, 0, 0)),
            pl.BlockSpec((H, D), lambda i: (0, 0)),
            pl.BlockSpec((H, D), lambda i: (0, 0)),
        ],
        out_specs=(
            pl.BlockSpec((B, D), lambda i: (i, 0)),
            pl.BlockSpec((H, B, D), lambda i: (0, i, 0)),
            pl.BlockSpec((H, B, 1), lambda i: (0, i, 0)),
            pl.BlockSpec((H, 1, B), lambda i: (0, 0, i)),
        ),
        out_shape=(
            jax.ShapeDtypeStruct((N, D), jnp.float32),
            jax.ShapeDtypeStruct((H, N, D), jnp.float32),
            jax.ShapeDtypeStruct((H, N, 1), jnp.float32),
            jax.ShapeDtypeStruct((H, 1, N), jnp.float32),
        ),
    )

    attn_last = pl.pallas_call(
        _attn_last_kernel,
        grid=(N // B,),
        in_specs=attn_in_specs,
        out_specs=pl.BlockSpec((B, D), lambda i: (i, 0)),
        out_shape=jax.ShapeDtypeStruct((N, D), jnp.float32),
    )

    gram = pl.pallas_call(
        _gram_kernel,
        grid=(N // GB,),
        in_specs=[
            pl.BlockSpec((N, D), lambda i: (0, 0)),
            pl.BlockSpec((GB, D), lambda i: (i, 0)),
        ],
        out_specs=pl.BlockSpec((N, GB), lambda i: (0, i)),
        out_shape=jax.ShapeDtypeStruct((N, N), jnp.float32),
    )

    h, f1, f2 = proj(x, W, a_src, a_dst)
    y1, h, f1, f2 = attn_mid(h, adj, x, f1, f2, W, a_src, a_dst)
    y2, h, f1, f2 = attn_mid(h, adj, y1, f1, f2, W, a_src, a_dst)
    y3 = attn_last(h, adj, y2, f1, f2)
    return gram(y3, y3)
